# baseline (speedup 1.0000x reference)
import functools

import numpy as np

import jax
import jax.numpy as jnp
from jax import lax
from jax.experimental import pallas as pl
from jax.experimental.pallas import tpu as pltpu
from jax.experimental.pallas import tpu_sc as plsc

NC = 2
NS = 16
LANES = 16
IDXW = 128
CHUNK = 8
TB = 14336


def _make_deg(NP, R):
    K2 = R // NS
    DCH = 12
    NB = 2
    nchunks = K2 // DCH
    G = nchunks // NB
    assert nchunks % NB == 0
    ZT = NP // NS
    mesh = plsc.VectorSubcoreMesh(core_axis_name="c", subcore_axis_name="s")

    @functools.partial(
        pl.kernel,
        out_type=jax.ShapeDtypeStruct((NC, NP), jnp.float32),
        mesh=mesh,
        compiler_params=pltpu.CompilerParams(use_tc_tiling_on_sc=False),
        scratch_types=[
            pltpu.VMEM_SHARED((NP,), jnp.float32),
            pltpu.VMEM((NB, DCH, IDXW), jnp.int32),
            pltpu.VMEM((IDXW,), jnp.float32),
            pltpu.VMEM((ZT,), jnp.float32),
            pltpu.SemaphoreType.DMA,
            pltpu.SemaphoreType.DMA,
            pltpu.SemaphoreType.DMA,
            pltpu.SemaphoreType.DMA,
        ],
    )
    def deg(idx_hbm, out_hbm, acc, sidx, ones, zbuf, sem0, sem1, isem0, isem1):
        c = lax.axis_index("c")
        s = lax.axis_index("s")
        sems = [sem0, sem1]
        isems = [isem0, isem1]

        def fill_ones(i, carry):
            ones[pl.ds(i * LANES, LANES)] = jnp.ones((LANES,), jnp.float32)
            return carry

        lax.fori_loop(0, IDXW // LANES, fill_ones, 0)

        def fill_z(i, carry):
            zbuf[pl.ds(i * LANES, LANES)] = jnp.zeros((LANES,), jnp.float32)
            return carry

        lax.fori_loop(0, ZT // LANES, fill_z, 0)
        pltpu.sync_copy(zbuf, acc.at[pl.ds(s * ZT, ZT)])
        plsc.subcore_barrier()

        base = s * K2
        icps = [
            pltpu.async_copy(
                idx_hbm.at[c, pl.ds(base + b * DCH, DCH), :], sidx.at[b],
                isems[b],
            )
            for b in range(NB)
        ]

        def group(g, carry):
            cps = []
            for b in range(NB):
                icps[b].wait()
                cps.append([
                    pltpu.async_copy(
                        ones, acc.at[sidx.at[b, j]], sems[b], add=True
                    )
                    for j in range(DCH)
                ])
            for b in range(NB):
                for cp in cps[b]:
                    cp.wait()
                nxt = jnp.minimum((g + 1) * NB + b, nchunks - 1)
                icps[b] = pltpu.async_copy(
                    idx_hbm.at[c, pl.ds(base + nxt * DCH, DCH), :], sidx.at[b],
                    isems[b],
                )
            return carry

        lax.fori_loop(0, G, group, 0)
        for b in range(NB):
            pltpu.make_async_copy(
                idx_hbm.at[c, pl.ds(0, DCH), :], sidx.at[b], isems[b]
            ).wait()
        plsc.subcore_barrier()
        pltpu.sync_copy(acc.at[pl.ds(s * ZT, ZT)], out_hbm.at[c, pl.ds(s * ZT, ZT)])

    return deg


def _make_agg(NP, R, K, D, CH, NB, DOUT=None):
    DOUT = D if DOUT is None else DOUT
    nchunks = K // CH
    G = nchunks // NB
    assert nchunks % NB == 0
    ZT = NP // NS
    mesh = plsc.VectorSubcoreMesh(core_axis_name="c", subcore_axis_name="s")

    @functools.partial(
        pl.kernel,
        out_type=jax.ShapeDtypeStruct((NC, NP, DOUT), jnp.float32),
        mesh=mesh,
        compiler_params=pltpu.CompilerParams(use_tc_tiling_on_sc=False),
        scratch_types=[
            pltpu.VMEM_SHARED((NP, D), jnp.float32),
            pltpu.VMEM((NB, 2, CH, IDXW), jnp.int32),
            pltpu.VMEM((NB, CH, IDXW, D), jnp.float32),
            pltpu.SemaphoreType.DMA,
            pltpu.SemaphoreType.DMA,
            pltpu.SemaphoreType.DMA,
            pltpu.SemaphoreType.DMA,
            pltpu.SemaphoreType.DMA,
            pltpu.SemaphoreType.DMA,
        ],
    )
    def agg(idx_hbm, table_hbm, zeros_hbm, out_hbm, acc, idxb, rows,
            gsem0, gsem1, ssem0, ssem1, isem0, isem1):
        c = lax.axis_index("c")
        s = lax.axis_index("s")
        wid = c * NS + s
        gsems = [gsem0, gsem1]
        ssems = [ssem0, ssem1]
        isems = [isem0, isem1]

        pltpu.sync_copy(zeros_hbm.at[pl.ds(s * ZT, ZT), :],
                        acc.at[pl.ds(s * ZT, ZT), :])
        plsc.subcore_barrier()

        base = wid * K

        def fire_idx(b, ci):
            return pltpu.async_copy(
                idx_hbm.at[:, pl.ds(base + ci * CH, CH), :], idxb.at[b],
                isems[b],
            )

        def fire_gathers(b):
            return [
                pltpu.async_copy(
                    table_hbm.at[idxb.at[b, 0, j]], rows.at[b, j], gsems[b]
                )
                for j in range(CH)
            ]

        icps = [fire_idx(b, b) for b in range(NB)]
        gcps = []
        for b in range(NB):
            icps[b].wait()
            gcps.append(fire_gathers(b))

        def group(g, carry):
            scps = []
            for b in range(NB):
                for cp in gcps[b]:
                    cp.wait()
                scps.append([
                    pltpu.async_copy(
                        rows.at[b, j], acc.at[idxb.at[b, 1, j]], ssems[b],
                        add=True,
                    )
                    for j in range(CH)
                ])
            for b in range(NB):
                for cp in scps[b]:
                    cp.wait()
                nxt = jnp.minimum((g + 1) * NB + b, nchunks - 1)
                icps[b] = fire_idx(b, nxt)
            for b in range(NB):
                icps[b].wait()
                gcps[b] = fire_gathers(b)
            return carry

        lax.fori_loop(0, G, group, 0)
        for b in range(NB):
            for j in range(CH):
                pltpu.make_async_copy(
                    table_hbm.at[idxb.at[b, 0, j]], rows.at[b, j], gsems[b]
                ).wait()
        plsc.subcore_barrier()
        pltpu.sync_copy(
            acc.at[pl.ds(s * ZT, ZT), pl.ds(0, DOUT)],
            out_hbm.at[c, pl.ds(s * ZT, ZT), :],
        )

    return agg


def _prep(cs, featsp):
    NP, DW = featsp.shape
    G = NP // TB

    def body(cs_ref, x_ref, t_ref):
        ns = lax.rsqrt(jnp.maximum(cs_ref[:], 1.0))
        t_ref[:] = x_ref[:] * ns[:, None]

    return pl.pallas_call(
        body,
        grid=(G,),
        in_specs=[
            pl.BlockSpec((TB,), lambda i: (i,)),
            pl.BlockSpec((TB, DW), lambda i: (i, 0)),
        ],
        out_specs=pl.BlockSpec((TB, DW), lambda i: (i, 0)),
        out_shape=jax.ShapeDtypeStruct((NP, DW), jnp.float32),
    )(cs, featsp)


def _mid(part1, cs, cd, W1p, b1, W2):
    _, NP, DW = part1.shape
    DO = W2.shape[1]
    G = NP // TB

    def body(m_ref, cs_ref, cd_ref, w1_ref, b_ref, w2_ref, t_ref):
        ns = lax.rsqrt(jnp.maximum(cs_ref[:], 1.0))
        nd = lax.rsqrt(jnp.maximum(cd_ref[:], 1.0))
        m = m_ref[0] + m_ref[1]
        h = jnp.dot(m, w1_ref[:], preferred_element_type=jnp.float32)
        x1 = jnp.maximum(h * nd[:, None] + b_ref[:][None, :], 0.0)
        y = jnp.dot(
            x1 * ns[:, None], w2_ref[:], preferred_element_type=jnp.float32
        )
        t_ref[:] = jnp.concatenate(
            [y, jnp.zeros((y.shape[0], 8 - DO), jnp.float32)], axis=1
        )

    return pl.pallas_call(
        body,
        grid=(G,),
        in_specs=[
            pl.BlockSpec((NC, TB, DW), lambda i: (0, i, 0)),
            pl.BlockSpec((TB,), lambda i: (i,)),
            pl.BlockSpec((TB,), lambda i: (i,)),
            pl.BlockSpec((DW, DW), lambda i: (0, 0)),
            pl.BlockSpec((DW,), lambda i: (0,)),
            pl.BlockSpec((DW, DO), lambda i: (0, 0)),
        ],
        out_specs=pl.BlockSpec((TB, 8), lambda i: (i, 0)),
        out_shape=jax.ShapeDtypeStruct((NP, 8), jnp.float32),
    )(part1, cs, cd, W1p, b1, W2)


def _fin(part2, cd, b2, N):
    _, NP, DP = part2.shape
    DO = b2.shape[0]
    G = -(-N // TB)

    def body(m_ref, cd_ref, b_ref, o_ref):
        nd = lax.rsqrt(jnp.maximum(cd_ref[:], 1.0))
        m = m_ref[0, :, :DO] + m_ref[1, :, :DO]
        o_ref[:] = m * nd[:, None] + b_ref[:][None, :]

    return pl.pallas_call(
        body,
        grid=(G,),
        in_specs=[
            pl.BlockSpec((NC, TB, DP), lambda i: (0, i, 0)),
            pl.BlockSpec((TB,), lambda i: (i,)),
            pl.BlockSpec((DO,), lambda i: (0,)),
        ],
        out_specs=pl.BlockSpec((TB, DO), lambda i: (i, 0)),
        out_shape=jax.ShapeDtypeStruct((N, DO), jnp.float32),
    )(part2, cd, b2)


def kernel(features, edge_index, W1, b1, W2, b2):
    N, DIN = features.shape
    E = edge_index.shape[1]
    DH = W1.shape[1]
    DO = W2.shape[1]

    rows = -(-E // IDXW)
    K = -(-rows // (NC * NS * 36)) * 36
    R = K * NC * NS
    Epad = R * IDXW
    NP = -(-(N + 1) // TB) * TB
    DW = 16

    pad_np = np.broadcast_to(
        (N + (np.arange(Epad - E) % 128)).astype(np.int32), (2, Epad - E)
    )
    idxp = jnp.concatenate([edge_index, jnp.asarray(pad_np)], axis=1)
    idx2 = idxp.reshape(2, R, IDXW)
    featsp = jnp.zeros((NP, DW), features.dtype).at[:N, :DIN].set(features)
    W1p = jnp.zeros((DW, DH), W1.dtype).at[:DIN, :].set(W1)

    cnts = _make_deg(NP, R)(idx2)
    xs = _prep(cnts[0], featsp)
    part1 = _make_agg(NP, R, K, DW, 6, 2)(idx2, xs, jnp.zeros((NP, DW), jnp.float32))
    table2 = _mid(part1, cnts[0], cnts[1], W1p, b1, W2)
    part2 = _make_agg(NP, R, K, 8, 12, 2)(
        idx2, table2, jnp.zeros((NP, 8), jnp.float32)
    )
    return _fin(part2, cnts[1], b2, N)
